# baseline (speedup 1.0000x reference)
import functools

import jax
import jax.numpy as jnp
from jax import lax
from jax.experimental import pallas as pl
from jax.experimental.pallas import tpu as pltpu
from jax.experimental.pallas import tpu_sc as plsc

VOCAB = 1000000
LENGTH = 200
DIM = 64
BATCH = 4096
VBLK = 256
NVBLK = VOCAB // VBLK
VTAIL = VOCAB - NVBLK * VBLK


def _positional_encoding(length, dim, n=10000):
    half_dim = dim // 2
    pos = jnp.arange(length, dtype=jnp.float32).reshape(-1, 1)
    i = jnp.arange(half_dim, dtype=jnp.float32).reshape(1, -1)
    denom = jnp.power(jnp.float32(n), -i / half_dim)
    args = pos * denom
    sin = jnp.expand_dims(jnp.sin(args), axis=-1)
    cos = jnp.expand_dims(jnp.cos(args), axis=-1)
    return jnp.concatenate([sin, cos], axis=-1).reshape(length, dim)


def _make_repack_kernel(num_cores, num_subcores):
    nw = num_cores * num_subcores

    mesh = plsc.VectorSubcoreMesh(core_axis_name="c", subcore_axis_name="s")

    @functools.partial(
        pl.kernel,
        mesh=mesh,
        out_type=jax.ShapeDtypeStruct((VOCAB, 128), jnp.float32),
        scratch_types=[
            pltpu.VMEM((2, DIM, VBLK), jnp.float32),
            pltpu.VMEM((2, VBLK, 128), jnp.float32),
            pltpu.VMEM((DIM, VTAIL), jnp.float32),
            pltpu.SemaphoreType.DMA,
            pltpu.SemaphoreType.DMA,
            pltpu.SemaphoreType.DMA,
        ],
        compiler_params=pltpu.CompilerParams(needs_layout_passes=False),
    )
    def repack(tt_hbm, tail_hbm, out_hbm, tin, tout, ttail_v, isem0, isem1, osem):
        wid = lax.axis_index("s") * num_cores + lax.axis_index("c")
        cnt = (NVBLK - 1 - wid) // nw + 1
        isems = (isem0, isem1)

        def in_copy(k, slot):
            v0 = (wid + (k * nw)) * VBLK
            return pltpu.make_async_copy(
                tt_hbm.at[:, pl.ds(v0, VBLK)], tin.at[slot], isems[slot]
            )

        def out_copy(k, slot):
            v0 = (wid + (k * nw)) * VBLK
            return pltpu.make_async_copy(
                tout.at[slot], out_hbm.at[pl.ds(v0, VBLK)], osem
            )

        in_copy(0, 0).start()
        iota = lax.iota(jnp.int32, 16)
        one = jnp.full((16,), 1, jnp.int32)
        js = [jnp.full((16,), j, jnp.int32) for j in range(8)]

        @pl.loop(0, (NVBLK // nw + 2) // 2)
        def _g(g):
            for half in range(2):
                k = g * 2 + half
                slot = half

                @pl.when(k < cnt)
                def _():
                    @pl.when(k + 1 < cnt)
                    def _():
                        in_copy(k + 1, 1 - slot).start()

                    in_copy(k, slot).wait()

                    @pl.when(k >= 2)
                    def _():
                        out_copy(k - 2, slot).wait()

                    @pl.loop(0, VBLK // 16)
                    def _q(q):
                        vidx = iota + q * 16
                        for d0 in range(0, DIM, 8):
                            db = jnp.full((16,), d0, jnp.int32)
                            vals = [
                                tin[slot, d0 + j, pl.ds(q * 16, 16)]
                                for j in range(8)
                            ]
                            for j in range(8):
                                plsc.store_scatter(
                                    tout.at[slot], [vidx, db + js[j]], vals[j]
                                )

                    out_copy(k, slot).start()

        out_copy(0, 0).wait()
        out_copy(0, 1).wait()

        @pl.when(wid == 0)
        def _tail():
            v0 = NVBLK * VBLK
            pltpu.sync_copy(tail_hbm, ttail_v)

            @pl.loop(0, VTAIL // 16)
            def _q(q):
                vidx = iota + q * 16
                dv = jnp.full((16,), 0, jnp.int32)
                for d in range(DIM):
                    vals = ttail_v[d, pl.ds(q * 16, 16)]
                    plsc.store_scatter(tout.at[0], [vidx, dv], vals)
                    dv = dv + one

            pltpu.sync_copy(tout.at[0, pl.ds(0, VTAIL)],
                            out_hbm.at[pl.ds(v0, VTAIL)])

    return repack


def _make_gather_kernel(num_cores, num_subcores):
    nw = num_cores * num_subcores
    bw = BATCH // nw
    nbt = BATCH // 128
    mesh = plsc.VectorSubcoreMesh(core_axis_name="c", subcore_axis_name="s")

    @functools.partial(
        pl.kernel,
        mesh=mesh,
        out_type=jax.ShapeDtypeStruct((LENGTH, DIM // 8, nbt, 8, 128), jnp.float32),
        scratch_types=[
            pltpu.VMEM((LENGTH, bw), jnp.int32),
            pltpu.VMEM((LENGTH, DIM), jnp.float32),
            pltpu.VMEM((3, bw, 128), jnp.float32),
            pltpu.VMEM((2, DIM // 8, 8, 128), jnp.float32),
            pltpu.SemaphoreType.DMA,
            pltpu.SemaphoreType.DMA,
            pltpu.SemaphoreType.DMA,
            pltpu.SemaphoreType.DMA,
        ],
        compiler_params=pltpu.CompilerParams(needs_layout_passes=False),
    )
    def sc_kernel(idx_hbm, table_hbm, pe_hbm, out_hbm,
                  idx_v, pe_v, rows_v, blk_v,
                  gsem0, gsem1, gsem2, osem):
        wid = lax.axis_index("s") * num_cores + lax.axis_index("c")
        b0 = wid * bw
        pltpu.sync_copy(idx_hbm.at[:, pl.ds(b0, bw)], idx_v)
        pltpu.sync_copy(pe_hbm, pe_v)

        gsems = (gsem0, gsem1, gsem2)

        def gather_copy(p, slot):
            return pltpu.make_async_copy(
                table_hbm.at[idx_v.at[p]],
                rows_v.at[slot],
                gsems[slot],
            )

        def block_write(p, slot):
            return pltpu.make_async_copy(
                blk_v.at[slot],
                out_hbm.at[p, :, wid],
                osem,
            )

        iota = lax.iota(jnp.int32, 16)
        dts = [(iota + 16 * q) // 8 for q in range(DIM // 16)]
        rs = [lax.rem(iota + 16 * q, 8) for q in range(DIM // 16)]

        def compute(p, slot, bslot):
            pes = [pe_v[p, pl.ds(16 * q, 16)] for q in range(DIM // 16)]

            @pl.loop(0, bw, unroll=4)
            def _c(c):
                cs = jnp.full((16,), c, jnp.int32)
                vals = [
                    rows_v[slot, c, pl.ds(16 * q, 16)]
                    for q in range(DIM // 16)
                ]
                for q in range(DIM // 16):
                    plsc.store_scatter(
                        blk_v.at[bslot], [dts[q], rs[q], cs], vals[q] + pes[q]
                    )

        for j in range(3):
            gather_copy(j, j).start()

        def body(p, slot, bslot):
            gather_copy(p, slot).wait()

            @pl.when(p + 3 < LENGTH)
            def _():
                gather_copy(p + 3, slot).start()

            @pl.when(p >= 2)
            def _():
                block_write(p - 2, bslot).wait()

            compute(p, slot, bslot)
            block_write(p, bslot).start()

        @pl.loop(0, (LENGTH - 2) // 6)
        def _pos(g):
            for j in range(6):
                p = g * 6 + j
                body(p, j % 3, j % 2)

        body(LENGTH - 2, (LENGTH - 2) % 3, (LENGTH - 2) % 2)
        body(LENGTH - 1, (LENGTH - 1) % 3, (LENGTH - 1) % 2)

        block_write(LENGTH - 2, 0).wait()
        block_write(LENGTH - 1, 1).wait()

    return sc_kernel


def kernel(inputs, table):
    pe = _positional_encoding(LENGTH, DIM)
    info = plsc.get_sparse_core_info()
    repack = _make_repack_kernel(info.num_cores, info.num_subcores)
    sc_kernel = _make_gather_kernel(info.num_cores, info.num_subcores)
    tt = jnp.transpose(table)
    tail = jnp.transpose(table[NVBLK * VBLK:, :])
    t128 = repack(tt, tail)
    idx_t = jnp.transpose(inputs.astype(jnp.int32))
    out5 = sc_kernel(idx_t, t128, pe)
    return out5.transpose(2, 4, 0, 1, 3).reshape(BATCH, LENGTH, DIM)


# speedup vs baseline: 1.0008x; 1.0008x over previous
import functools

import jax
import jax.numpy as jnp
from jax import lax
from jax.experimental import pallas as pl
from jax.experimental.pallas import tpu as pltpu
from jax.experimental.pallas import tpu_sc as plsc

VOCAB = 1000000
LENGTH = 200
DIM = 64
BATCH = 4096
VBLK = 256
NVBLK = VOCAB // VBLK
VTAIL = VOCAB - NVBLK * VBLK


def _positional_encoding(length, dim, n=10000):
    half_dim = dim // 2
    pos = jnp.arange(length, dtype=jnp.float32).reshape(-1, 1)
    i = jnp.arange(half_dim, dtype=jnp.float32).reshape(1, -1)
    denom = jnp.power(jnp.float32(n), -i / half_dim)
    args = pos * denom
    sin = jnp.expand_dims(jnp.sin(args), axis=-1)
    cos = jnp.expand_dims(jnp.cos(args), axis=-1)
    return jnp.concatenate([sin, cos], axis=-1).reshape(length, dim)


def _make_repack_kernel(num_cores, num_subcores):
    nw = num_cores * num_subcores

    mesh = plsc.VectorSubcoreMesh(core_axis_name="c", subcore_axis_name="s")

    @functools.partial(
        pl.kernel,
        mesh=mesh,
        out_type=jax.ShapeDtypeStruct((VOCAB, 128), jnp.float32),
        scratch_types=[
            pltpu.VMEM((2, DIM, VBLK), jnp.float32),
            pltpu.VMEM((2, VBLK, 128), jnp.float32),
            pltpu.VMEM((DIM, VTAIL), jnp.float32),
            pltpu.SemaphoreType.DMA,
            pltpu.SemaphoreType.DMA,
            pltpu.SemaphoreType.DMA,
        ],
        compiler_params=pltpu.CompilerParams(needs_layout_passes=False),
    )
    def repack(tt_hbm, tail_hbm, out_hbm, tin, tout, ttail_v, isem0, isem1, osem):
        wid = lax.axis_index("s") * num_cores + lax.axis_index("c")
        cnt = (NVBLK - 1 - wid) // nw + 1
        isems = (isem0, isem1)

        def in_copy(k, slot):
            v0 = (wid + (k * nw)) * VBLK
            return pltpu.make_async_copy(
                tt_hbm.at[:, pl.ds(v0, VBLK)], tin.at[slot], isems[slot]
            )

        def out_copy(k, slot):
            v0 = (wid + (k * nw)) * VBLK
            return pltpu.make_async_copy(
                tout.at[slot], out_hbm.at[pl.ds(v0, VBLK)], osem
            )

        in_copy(0, 0).start()
        iota = lax.iota(jnp.int32, 16)
        one = jnp.full((16,), 1, jnp.int32)
        js = [jnp.full((16,), j, jnp.int32) for j in range(8)]

        @pl.loop(0, (NVBLK // nw + 2) // 2)
        def _g(g):
            for half in range(2):
                k = g * 2 + half
                slot = half

                @pl.when(k < cnt)
                def _():
                    @pl.when(k + 1 < cnt)
                    def _():
                        in_copy(k + 1, 1 - slot).start()

                    in_copy(k, slot).wait()

                    @pl.when(k >= 2)
                    def _():
                        out_copy(k - 2, slot).wait()

                    @pl.loop(0, VBLK // 16)
                    def _q(q):
                        vidx = iota + q * 16
                        for d0 in range(0, DIM, 8):
                            db = jnp.full((16,), d0, jnp.int32)
                            vals = [
                                tin[slot, d0 + j, pl.ds(q * 16, 16)]
                                for j in range(8)
                            ]
                            for j in range(8):
                                plsc.store_scatter(
                                    tout.at[slot], [vidx, db + js[j]], vals[j]
                                )

                    out_copy(k, slot).start()

        out_copy(0, 0).wait()
        out_copy(0, 1).wait()

        @pl.when(wid == 0)
        def _tail():
            v0 = NVBLK * VBLK
            pltpu.sync_copy(tail_hbm, ttail_v)

            @pl.loop(0, VTAIL // 16)
            def _q(q):
                vidx = iota + q * 16
                dv = jnp.full((16,), 0, jnp.int32)
                for d in range(DIM):
                    vals = ttail_v[d, pl.ds(q * 16, 16)]
                    plsc.store_scatter(tout.at[0], [vidx, dv], vals)
                    dv = dv + one

            pltpu.sync_copy(tout.at[0, pl.ds(0, VTAIL)],
                            out_hbm.at[pl.ds(v0, VTAIL)])

    return repack


def _make_gather_kernel(num_cores, num_subcores):
    nw = num_cores * num_subcores
    bw = BATCH // nw
    nbt = BATCH // 128
    mesh = plsc.VectorSubcoreMesh(core_axis_name="c", subcore_axis_name="s")

    @functools.partial(
        pl.kernel,
        mesh=mesh,
        out_type=jax.ShapeDtypeStruct((LENGTH, DIM // 8, nbt, 8, 128), jnp.float32),
        scratch_types=[
            pltpu.VMEM((LENGTH, bw), jnp.int32),
            pltpu.VMEM((LENGTH, DIM), jnp.float32),
            pltpu.VMEM((3, bw, 128), jnp.float32),
            pltpu.VMEM((2, DIM // 8, 8, 128), jnp.float32),
            pltpu.SemaphoreType.DMA,
            pltpu.SemaphoreType.DMA,
            pltpu.SemaphoreType.DMA,
            pltpu.SemaphoreType.DMA,
        ],
        compiler_params=pltpu.CompilerParams(needs_layout_passes=False),
    )
    def sc_kernel(idx_hbm, table_hbm, pe_hbm, out_hbm,
                  idx_v, pe_v, rows_v, blk_v,
                  gsem0, gsem1, gsem2, osem):
        wid = lax.axis_index("s") * num_cores + lax.axis_index("c")
        b0 = wid * bw
        pltpu.sync_copy(idx_hbm.at[:, pl.ds(b0, bw)], idx_v)
        pltpu.sync_copy(pe_hbm, pe_v)

        gsems = (gsem0, gsem1, gsem2)

        def gather_copy(p, slot):
            return pltpu.make_async_copy(
                table_hbm.at[idx_v.at[p]],
                rows_v.at[slot],
                gsems[slot],
            )

        def block_write(p, slot):
            return pltpu.make_async_copy(
                blk_v.at[slot],
                out_hbm.at[p, :, wid],
                osem,
            )

        iota = lax.iota(jnp.int32, 16)
        dts = [(iota + 16 * q) // 8 for q in range(DIM // 16)]
        rs = [lax.rem(iota + 16 * q, 8) for q in range(DIM // 16)]

        def compute(p, slot, bslot):
            pes = [pe_v[p, pl.ds(16 * q, 16)] for q in range(DIM // 16)]

            @pl.loop(0, bw, unroll=4)
            def _c(c):
                cs = jnp.full((16,), c, jnp.int32)
                vals = [
                    rows_v[slot, c, pl.ds(16 * q, 16)]
                    for q in range(DIM // 16)
                ]
                for q in range(DIM // 16):
                    plsc.store_scatter(
                        blk_v.at[bslot], [dts[q], rs[q], cs], vals[q] + pes[q]
                    )

        for j in range(3):
            gather_copy(j, j).start()

        def body(p, slot, bslot):
            gather_copy(p, slot).wait()

            @pl.when(p >= 2)
            def _():
                block_write(p - 2, bslot).wait()

            compute(p, slot, bslot)

            @pl.when(p + 3 < LENGTH)
            def _():
                gather_copy(p + 3, slot).start()

            block_write(p, bslot).start()

        @pl.loop(0, (LENGTH - 2) // 6)
        def _pos(g):
            for j in range(6):
                p = g * 6 + j
                body(p, j % 3, j % 2)

        body(LENGTH - 2, (LENGTH - 2) % 3, (LENGTH - 2) % 2)
        body(LENGTH - 1, (LENGTH - 1) % 3, (LENGTH - 1) % 2)

        block_write(LENGTH - 2, 0).wait()
        block_write(LENGTH - 1, 1).wait()

    return sc_kernel


def kernel(inputs, table):
    pe = _positional_encoding(LENGTH, DIM)
    info = plsc.get_sparse_core_info()
    repack = _make_repack_kernel(info.num_cores, info.num_subcores)
    sc_kernel = _make_gather_kernel(info.num_cores, info.num_subcores)
    tt = jnp.transpose(table)
    tail = jnp.transpose(table[NVBLK * VBLK:, :])
    t128 = repack(tt, tail)
    idx_t = jnp.transpose(inputs.astype(jnp.int32))
    out5 = sc_kernel(idx_t, t128, pe)
    return out5.transpose(2, 4, 0, 1, 3).reshape(BATCH, LENGTH, DIM)
